# baseline (speedup 1.0000x reference)
import functools
import math

import numpy as np
import jax
import jax.numpy as jnp
from jax import lax
from jax.experimental import pallas as pl
from jax.experimental.pallas import tpu as pltpu
from jax.experimental.pallas import tpu_sc as plsc

_M32 = 0xFFFFFFFF


def _py_threefry2x32(k1, k2, x0, x1):
    def rotl(x, d):
        return ((x << d) | (x >> (32 - d))) & _M32

    ks0, ks1 = k1, k2
    ks2 = k1 ^ k2 ^ 0x1BD11BDA
    x0 = (x0 + ks0) & _M32
    x1 = (x1 + ks1) & _M32
    rots = ((13, 15, 26, 6), (17, 29, 16, 24))

    def r4(x0, x1, rs):
        for r in rs:
            x0 = (x0 + x1) & _M32
            x1 = rotl(x1, r)
            x1 = x0 ^ x1
        return x0, x1

    x0, x1 = r4(x0, x1, rots[0]); x0 = (x0 + ks1) & _M32; x1 = (x1 + ks2 + 1) & _M32
    x0, x1 = r4(x0, x1, rots[1]); x0 = (x0 + ks2) & _M32; x1 = (x1 + ks0 + 2) & _M32
    x0, x1 = r4(x0, x1, rots[0]); x0 = (x0 + ks0) & _M32; x1 = (x1 + ks1 + 3) & _M32
    x0, x1 = r4(x0, x1, rots[1]); x0 = (x0 + ks1) & _M32; x1 = (x1 + ks2 + 4) & _M32
    x0, x1 = r4(x0, x1, rots[0]); x0 = (x0 + ks2) & _M32; x1 = (x1 + ks0 + 5) & _M32
    return x0, x1


_SK1, _SK2 = _py_threefry2x32(0, 42, 0, 1)

_TINY = np.float32(np.finfo(np.float32).tiny)
_CUTOFF = 20.45
_E_LB = np.float32(1.178e-7)
_MBIG = np.float32(3e38)
_KBIG = np.int32(0x7FFFFFFF)
_KMAX = 4096
_CH = 8


def _k1a_body(prev_ref, prevT_ref, noise_ref, noiseT_ref, ctrl_ref, ctrlT_ref,
              A_ref, AT_ref, B_ref, BT_ref, CT_ref, obsT_ref, lwp_ref,
              spn_ref, a_ref):
    prev = prev_ref[0]
    ctrl = ctrl_ref[0]
    cb = jnp.dot(ctrl, B_ref[...], preferred_element_type=jnp.float32)
    spn = jnp.dot(prev, A_ref[...], preferred_element_type=jnp.float32) + cb
    spn = spn + noise_ref[0]
    spn_ref[0] = spn

    prevT = prevT_ref[0]
    ctrlT = ctrlT_ref[0]
    cbT = jnp.dot(BT_ref[...], ctrlT, preferred_element_type=jnp.float32)
    spT = jnp.dot(AT_ref[...], prevT, preferred_element_type=jnp.float32) + cbT
    spT = spT + noiseT_ref[0]
    diffT = jnp.dot(CT_ref[...], spT, preferred_element_type=jnp.float32) \
        - obsT_ref[0]
    s = jnp.sum(diffT * diffT, axis=0, keepdims=True)
    a_ref[0] = lwp_ref[0] + (-0.5) * s


def _k1b_body(a_ref, lw_ref):
    a = a_ref[0]
    amax = jnp.max(a)
    se = jnp.sum(jnp.exp(a - amax))
    lse = jnp.log(se) + amax
    lw_ref[0] = a - lse


def _k1c_body(lw_ref, spn_ref, best_ref):
    mb = pl.program_id(1)

    @pl.when(mb == 0)
    def _():
        best_ref[0] = jnp.zeros_like(best_ref[0])

    best_ref[0] += jnp.dot(jnp.exp(lw_ref[0]), spn_ref[0],
                           preferred_element_type=jnp.float32)


def _tf_bits(hi, lo):
    ks0 = np.uint32(_SK1)
    ks1 = np.uint32(_SK2)
    ks2 = np.uint32(_SK1 ^ _SK2 ^ 0x1BD11BDA)

    def r4(x0, x1, rs):
        for r in rs:
            x0 = x0 + x1
            x1 = (x1 << r) | (x1 >> (32 - r))
            x1 = x0 ^ x1
        return x0, x1

    x0 = hi + ks0
    x1 = lo + ks1
    x0, x1 = r4(x0, x1, (13, 15, 26, 6))
    x0 = x0 + ks1; x1 = x1 + (ks2 + np.uint32(1))
    x0, x1 = r4(x0, x1, (17, 29, 16, 24))
    x0 = x0 + ks2; x1 = x1 + (ks0 + np.uint32(2))
    x0, x1 = r4(x0, x1, (13, 15, 26, 6))
    x0 = x0 + ks0; x1 = x1 + (ks1 + np.uint32(3))
    x0, x1 = r4(x0, x1, (17, 29, 16, 24))
    x0 = x0 + ks1; x1 = x1 + (ks2 + np.uint32(4))
    x0, x1 = r4(x0, x1, (13, 15, 26, 6))
    x0 = x0 + ks2; x1 = x1 + (ks0 + np.uint32(5))
    return x0 ^ x1


def _make_k2_body(n_rows, m_cat, kmax, tiles_per_prog):
    nch = kmax // _CH
    sh_m = int(math.log2(m_cat))
    assert (1 << sh_m) == m_cat
    nu = np.uint32(n_rows)

    def body(karr_ref, warr_ref, thr_ref, out_ref, mout_ref):
        j = pl.program_id(0)
        sb = pl.program_id(1)
        ju = lax.convert_element_type(j, jnp.uint32)

        for t in range(tiles_per_prog):
            i0 = sb * (tiles_per_prog * 128) + t * 128
            ii = lax.broadcasted_iota(jnp.int32, (_CH, 128), 1) + i0
            iu = lax.convert_element_type(ii, jnp.uint32)
            q = iu * nu + ju
            lo_base = q << sh_m
            hi = q >> (32 - sh_m)

            def cond(carry):
                c, mb, kb = carry
                return c < nch

            def step(carry):
                c, mb, kb = carry
                kcol = karr_ref[0, pl.ds(c * _CH, _CH), :]
                wcol = warr_ref[0, pl.ds(c * _CH, _CH), :]
                lo = lo_base + lax.convert_element_type(kcol, jnp.uint32)
                bits = _tf_bits(hi, lo)
                sh = bits >> 9
                fl = lax.bitcast_convert_type(sh | np.uint32(0x3F800000),
                                              jnp.float32) - 1.0
                u = jnp.where(sh == np.uint32(0), _TINY, fl)
                e = -jnp.log(u)
                mm = e * wcol
                kbr = jnp.broadcast_to(kcol, (_CH, 128))
                upd = (mm < mb) | ((mm == mb) & (kbr < kb))
                mb = jnp.where(upd, mm, mb)
                kb = jnp.where(upd, kbr, kb)
                tval = thr_ref[0, 0, c]
                done = jnp.all(mb < tval)
                c = jnp.where(done, nch, c + 1)
                return c, mb, kb

            mb0 = jnp.full((_CH, 128), _MBIG, jnp.float32)
            kb0 = jnp.full((_CH, 128), _KBIG, jnp.int32)
            _, mb, kb = lax.while_loop(cond, step, (jnp.int32(0), mb0, kb0))

            mmin = jnp.min(mb, axis=0, keepdims=True)
            kmin = jnp.min(jnp.where(mb == mmin, kb, _KBIG), axis=0,
                           keepdims=True)
            out_ref[0, :, pl.ds(t * 128, 128)] = kmin
            mout_ref[0, :, pl.ds(t * 128, 128)] = mmin

    return body


def _run_k2(karr, warr, thr, n_rows, m_cat, kmax):
    tiles_per_prog = 8
    ts = tiles_per_prog * 128
    nsb = m_cat // ts
    nch = kmax // _CH
    body = _make_k2_body(n_rows, m_cat, kmax, tiles_per_prog)
    kout, mout = pl.pallas_call(
        body,
        grid=(n_rows, nsb),
        in_specs=[
            pl.BlockSpec((1, kmax, 1), lambda j, sb: (j, 0, 0)),
            pl.BlockSpec((1, kmax, 1), lambda j, sb: (j, 0, 0)),
            pl.BlockSpec((1, 1, nch), lambda j, sb: (j, 0, 0),
                         memory_space=pltpu.SMEM),
        ],
        out_specs=[
            pl.BlockSpec((1, 1, ts), lambda j, sb: (j * nsb + sb, 0, 0)),
            pl.BlockSpec((1, 1, ts), lambda j, sb: (j * nsb + sb, 0, 0)),
        ],
        out_shape=[
            jax.ShapeDtypeStruct((n_rows * nsb, 1, ts), jnp.int32),
            jax.ShapeDtypeStruct((n_rows * nsb, 1, ts), jnp.float32),
        ],
    )(karr, warr, thr.reshape(n_rows, 1, nch))
    return kout.reshape(n_rows, m_cat), mout.reshape(n_rows, m_cat)


def _sc_gather(table, idx2d, total_rows, d):
    info = plsc.get_sparse_core_info()
    nw = info.num_cores * info.num_subcores
    b_per_w = total_rows // nw
    chrows = 2048
    nchunks = b_per_w // chrows
    rows_per_chunk_idx = chrows // 128
    mesh = plsc.VectorSubcoreMesh(core_axis_name="c", subcore_axis_name="s")

    @functools.partial(
        pl.kernel, mesh=mesh,
        compiler_params=pltpu.CompilerParams(use_tc_tiling_on_sc=False),
        out_type=jax.ShapeDtypeStruct((total_rows, d), jnp.float32),
        scratch_types=[
            pltpu.VMEM((rows_per_chunk_idx, 128), jnp.int32),
            pltpu.VMEM((chrows, d), jnp.float32),
            pltpu.SemaphoreType.DMA,
        ],
    )
    def k3(table_hbm, idx_hbm, out_hbm, idx_v, rows_v, sem):
        wid = lax.axis_index("s") * info.num_cores + lax.axis_index("c")
        base_row = wid * b_per_w
        base_irow = wid * (b_per_w // 128)

        def chunk(ch, carry):
            pltpu.sync_copy(
                idx_hbm.at[pl.ds(base_irow + ch * rows_per_chunk_idx,
                                 rows_per_chunk_idx)], idx_v)
            handles = []
            for r in range(rows_per_chunk_idx):
                handles.append(pltpu.async_copy(
                    table_hbm.at[idx_v.at[r]],
                    rows_v.at[pl.ds(r * 128, 128)], sem))
            for h in handles:
                h.wait()
            pltpu.sync_copy(rows_v,
                            out_hbm.at[pl.ds(base_row + ch * chrows, chrows)])
            return carry

        lax.fori_loop(0, nchunks, chunk, 0)

    return k3(table, idx2d)


def kernel(states_prev, log_weights_prev, observations, controls, A, B, C):
    n, m, d = states_prev.shape
    do = observations.shape[1]
    base_key = jax.random.key(42)
    noise = jax.random.normal(jax.random.fold_in(base_key, 0),
                              states_prev.shape, jnp.float32) * 0.1

    prevT = jnp.swapaxes(states_prev, 1, 2)
    noiseT = jnp.swapaxes(noise, 1, 2)

    mblk = 4096
    nmb = m // mblk
    dc = controls.shape[1]

    spn, a3 = pl.pallas_call(
        _k1a_body,
        grid=(n, nmb),
        in_specs=[
            pl.BlockSpec((1, mblk, d), lambda j, mb: (j, mb, 0)),
            pl.BlockSpec((1, d, mblk), lambda j, mb: (j, 0, mb)),
            pl.BlockSpec((1, mblk, d), lambda j, mb: (j, mb, 0)),
            pl.BlockSpec((1, d, mblk), lambda j, mb: (j, 0, mb)),
            pl.BlockSpec((1, 1, dc), lambda j, mb: (j, 0, 0)),
            pl.BlockSpec((1, dc, 1), lambda j, mb: (j, 0, 0)),
            pl.BlockSpec(A.shape, lambda j, mb: (0, 0)),
            pl.BlockSpec(A.shape, lambda j, mb: (0, 0)),
            pl.BlockSpec(B.shape, lambda j, mb: (0, 0)),
            pl.BlockSpec((B.shape[1], B.shape[0]), lambda j, mb: (0, 0)),
            pl.BlockSpec((do, d), lambda j, mb: (0, 0)),
            pl.BlockSpec((1, do, 1), lambda j, mb: (j, 0, 0)),
            pl.BlockSpec((1, 1, mblk), lambda j, mb: (j, 0, mb)),
        ],
        out_specs=[
            pl.BlockSpec((1, mblk, d), lambda j, mb: (j, mb, 0)),
            pl.BlockSpec((1, 1, mblk), lambda j, mb: (j, 0, mb)),
        ],
        out_shape=[
            jax.ShapeDtypeStruct((n, m, d), jnp.float32),
            jax.ShapeDtypeStruct((n, 1, m), jnp.float32),
        ],
    )(states_prev, prevT, noise, noiseT,
      controls.reshape(n, 1, -1), controls.reshape(n, -1, 1), A, A.T,
      B, B.T, C.T, observations.reshape(n, do, 1),
      log_weights_prev.reshape(n, 1, m))

    lw3 = pl.pallas_call(
        _k1b_body,
        grid=(n,),
        in_specs=[pl.BlockSpec((1, 1, m), lambda j: (j, 0, 0))],
        out_specs=pl.BlockSpec((1, 1, m), lambda j: (j, 0, 0)),
        out_shape=jax.ShapeDtypeStruct((n, 1, m), jnp.float32),
    )(a3)

    best3 = pl.pallas_call(
        _k1c_body,
        grid=(n, nmb),
        in_specs=[
            pl.BlockSpec((1, 1, mblk), lambda j, mb: (j, 0, mb)),
            pl.BlockSpec((1, mblk, d), lambda j, mb: (j, mb, 0)),
        ],
        out_specs=pl.BlockSpec((1, 1, d), lambda j, mb: (j, 0, 0)),
        out_shape=jax.ShapeDtypeStruct((n, 1, d), jnp.float32),
    )(lw3, spn)

    lw = lw3.reshape(n, m)
    best_states = best3.reshape(n, d)

    lwmax = jnp.max(lw, axis=1, keepdims=True)
    mask = lw >= (lwmax - _CUTOFF)
    fits = jnp.max(jnp.sum(mask, axis=1)) <= _KMAX

    def fast_path(_):
        masked = jnp.where(mask, lw, -jnp.inf)
        vals, inds = lax.top_k(masked, _KMAX)
        w = jnp.exp(-vals)
        thr = jnp.concatenate(
            [w[:, _CH::_CH] * _E_LB,
             jnp.full((n, 1), jnp.inf, jnp.float32)], axis=1)
        karr = inds.astype(jnp.int32).reshape(n, _KMAX, 1)
        warr = w.reshape(n, _KMAX, 1)
        kb, _ = _run_k2(karr, warr, thr, n, m, _KMAX)
        return kb

    def slow_path(_):
        thr = jnp.full((n, _KMAX // _CH), -jnp.inf, jnp.float32)
        kb = jnp.full((n, m), _KBIG, jnp.int32)
        mb = jnp.full((n, m), jnp.inf, jnp.float32)
        for s in range(m // _KMAX):
            karr = (lax.broadcasted_iota(jnp.int32, (n, _KMAX, 1), 1)
                    + s * _KMAX)
            warr = jnp.exp(-lax.dynamic_slice_in_dim(lw, s * _KMAX, _KMAX,
                                                     axis=1))
            ks, ms = _run_k2(karr, warr.reshape(n, _KMAX, 1), thr, n, m,
                             _KMAX)
            upd = (ms < mb) | ((ms == mb) & (ks < kb))
            kb = jnp.where(upd, ks, kb)
            mb = jnp.where(upd, ms, mb)
        return kb

    kidx = lax.cond(fits, fast_path, slow_path, operand=None)
    gidx = kidx + jnp.arange(n, dtype=jnp.int32)[:, None] * m

    table = spn.reshape(n * m, d)
    idx2d = gidx.reshape((n * m) // 128, 128)
    states = _sc_gather(table, idx2d, n * m, d).reshape(n, m, d)

    log_weights = jnp.zeros((n, m), jnp.float32) - jnp.log(float(m))
    return best_states, states, log_weights


# speedup vs baseline: 6.4798x; 6.4798x over previous
import functools
import math

import numpy as np
import jax
import jax.numpy as jnp
from jax import lax
from jax.experimental import pallas as pl
from jax.experimental.pallas import tpu as pltpu
from jax.experimental.pallas import tpu_sc as plsc

_M32 = 0xFFFFFFFF


def _py_threefry2x32(k1, k2, x0, x1):
    def rotl(x, d):
        return ((x << d) | (x >> (32 - d))) & _M32

    ks0, ks1 = k1, k2
    ks2 = k1 ^ k2 ^ 0x1BD11BDA
    x0 = (x0 + ks0) & _M32
    x1 = (x1 + ks1) & _M32
    rots = ((13, 15, 26, 6), (17, 29, 16, 24))

    def r4(x0, x1, rs):
        for r in rs:
            x0 = (x0 + x1) & _M32
            x1 = rotl(x1, r)
            x1 = x0 ^ x1
        return x0, x1

    x0, x1 = r4(x0, x1, rots[0]); x0 = (x0 + ks1) & _M32; x1 = (x1 + ks2 + 1) & _M32
    x0, x1 = r4(x0, x1, rots[1]); x0 = (x0 + ks2) & _M32; x1 = (x1 + ks0 + 2) & _M32
    x0, x1 = r4(x0, x1, rots[0]); x0 = (x0 + ks0) & _M32; x1 = (x1 + ks1 + 3) & _M32
    x0, x1 = r4(x0, x1, rots[1]); x0 = (x0 + ks1) & _M32; x1 = (x1 + ks2 + 4) & _M32
    x0, x1 = r4(x0, x1, rots[0]); x0 = (x0 + ks2) & _M32; x1 = (x1 + ks0 + 5) & _M32
    return x0, x1


_SK1, _SK2 = _py_threefry2x32(0, 42, 0, 1)

_TINY = np.float32(np.finfo(np.float32).tiny)
_CUTOFF = 20.45
_E_LB = np.float32(1.178e-7)
_MBIG = np.float32(3e38)
_KBIG = np.int32(0x7FFFFFFF)
_KMAX = 4096
_CH = 8


def _k1a_body(prev_ref, prevT_ref, noise_ref, noiseT_ref, ctrl_ref, ctrlT_ref,
              A_ref, AT_ref, B_ref, BT_ref, CT_ref, obsT_ref, lwp_ref,
              spn_ref, a_ref):
    prev = prev_ref[0]
    ctrl = ctrl_ref[0]
    cb = jnp.dot(ctrl, B_ref[...], preferred_element_type=jnp.float32)
    spn = jnp.dot(prev, A_ref[...], preferred_element_type=jnp.float32) + cb
    spn = spn + noise_ref[0]
    spn_ref[0] = spn

    prevT = prevT_ref[0]
    ctrlT = ctrlT_ref[0]
    cbT = jnp.dot(BT_ref[...], ctrlT, preferred_element_type=jnp.float32)
    spT = jnp.dot(AT_ref[...], prevT, preferred_element_type=jnp.float32) + cbT
    spT = spT + noiseT_ref[0]
    diffT = jnp.dot(CT_ref[...], spT, preferred_element_type=jnp.float32) \
        - obsT_ref[0]
    s = jnp.sum(diffT * diffT, axis=0, keepdims=True)
    a_ref[0] = lwp_ref[0] + (-0.5) * s


def _k1b_body(a_ref, lw_ref):
    a = a_ref[0]
    amax = jnp.max(a)
    se = jnp.sum(jnp.exp(a - amax))
    lse = jnp.log(se) + amax
    lw_ref[0] = a - lse


def _k1c_body(lw_ref, spn_ref, best_ref):
    mb = pl.program_id(1)

    @pl.when(mb == 0)
    def _():
        best_ref[0] = jnp.zeros_like(best_ref[0])

    best_ref[0] += jnp.dot(jnp.exp(lw_ref[0]), spn_ref[0],
                           preferred_element_type=jnp.float32)


def _tf_bits(hi, lo):
    ks0 = np.uint32(_SK1)
    ks1 = np.uint32(_SK2)
    ks2 = np.uint32(_SK1 ^ _SK2 ^ 0x1BD11BDA)

    def r4(x0, x1, rs):
        for r in rs:
            x0 = x0 + x1
            x1 = (x1 << r) | (x1 >> (32 - r))
            x1 = x0 ^ x1
        return x0, x1

    x0 = hi + ks0
    x1 = lo + ks1
    x0, x1 = r4(x0, x1, (13, 15, 26, 6))
    x0 = x0 + ks1; x1 = x1 + (ks2 + np.uint32(1))
    x0, x1 = r4(x0, x1, (17, 29, 16, 24))
    x0 = x0 + ks2; x1 = x1 + (ks0 + np.uint32(2))
    x0, x1 = r4(x0, x1, (13, 15, 26, 6))
    x0 = x0 + ks0; x1 = x1 + (ks1 + np.uint32(3))
    x0, x1 = r4(x0, x1, (17, 29, 16, 24))
    x0 = x0 + ks1; x1 = x1 + (ks2 + np.uint32(4))
    x0, x1 = r4(x0, x1, (13, 15, 26, 6))
    x0 = x0 + ks2; x1 = x1 + (ks0 + np.uint32(5))
    return x0 ^ x1


def _make_k2_body(n_rows, m_cat, kmax, tiles_per_prog):
    nch = kmax // _CH
    sh_m = int(math.log2(m_cat))
    assert (1 << sh_m) == m_cat
    nu = np.uint32(n_rows)

    def body(karr_ref, warr_ref, thr_ref, out_ref, mout_ref):
        j = pl.program_id(0)
        sb = pl.program_id(1)
        ju = lax.convert_element_type(j, jnp.uint32)

        ts = tiles_per_prog * 128
        ii = lax.broadcasted_iota(jnp.int32, (_CH, ts), 1) + sb * ts
        iu = lax.convert_element_type(ii, jnp.uint32)
        q = iu * nu + ju
        lo_base = q << sh_m
        hi = q >> (32 - sh_m)

        def cond(carry):
            c, mb, kb = carry
            return c < nch

        def step(carry):
            c, mb, kb = carry
            kcol = karr_ref[0, pl.ds(c * _CH, _CH), :]
            wcol = warr_ref[0, pl.ds(c * _CH, _CH), :]
            lo = lo_base + lax.convert_element_type(kcol, jnp.uint32)
            bits = _tf_bits(hi, lo)
            sh = bits >> 9
            fl = lax.bitcast_convert_type(sh | np.uint32(0x3F800000),
                                          jnp.float32) - 1.0
            u = jnp.where(sh == np.uint32(0), _TINY, fl)
            e = -jnp.log(u)
            mm = e * wcol
            kbr = jnp.broadcast_to(kcol, (_CH, ts))
            upd = (mm < mb) | ((mm == mb) & (kbr < kb))
            mb = jnp.where(upd, mm, mb)
            kb = jnp.where(upd, kbr, kb)
            tval = thr_ref[0, 0, c]
            done = jnp.all(mb < tval)
            c = jnp.where(done, nch, c + 1)
            return c, mb, kb

        mb0 = jnp.full((_CH, ts), _MBIG, jnp.float32)
        kb0 = jnp.full((_CH, ts), _KBIG, jnp.int32)
        _, mb, kb = lax.while_loop(cond, step, (jnp.int32(0), mb0, kb0))

        mmin = jnp.min(mb, axis=0, keepdims=True)
        kmin = jnp.min(jnp.where(mb == mmin, kb, _KBIG), axis=0,
                       keepdims=True)
        out_ref[0, :, :] = kmin
        mout_ref[0, :, :] = mmin

    return body


def _run_k2(karr, warr, thr, n_rows, m_cat, kmax):
    tiles_per_prog = 8
    ts = tiles_per_prog * 128
    nsb = m_cat // ts
    nch = kmax // _CH
    body = _make_k2_body(n_rows, m_cat, kmax, tiles_per_prog)
    kout, mout = pl.pallas_call(
        body,
        grid=(n_rows, nsb),
        in_specs=[
            pl.BlockSpec((1, kmax, 1), lambda j, sb: (j, 0, 0)),
            pl.BlockSpec((1, kmax, 1), lambda j, sb: (j, 0, 0)),
            pl.BlockSpec((1, 1, nch), lambda j, sb: (j, 0, 0),
                         memory_space=pltpu.SMEM),
        ],
        out_specs=[
            pl.BlockSpec((1, 1, ts), lambda j, sb: (j * nsb + sb, 0, 0)),
            pl.BlockSpec((1, 1, ts), lambda j, sb: (j * nsb + sb, 0, 0)),
        ],
        out_shape=[
            jax.ShapeDtypeStruct((n_rows * nsb, 1, ts), jnp.int32),
            jax.ShapeDtypeStruct((n_rows * nsb, 1, ts), jnp.float32),
        ],
    )(karr, warr, thr.reshape(n_rows, 1, nch))
    return kout.reshape(n_rows, m_cat), mout.reshape(n_rows, m_cat)


def _sc_gather(table, idx2d, total_rows, d):
    info = plsc.get_sparse_core_info()
    nw = info.num_cores * info.num_subcores
    b_per_w = total_rows // nw
    chrows = 2048
    nchunks = b_per_w // chrows
    rows_per_chunk_idx = chrows // 128
    mesh = plsc.VectorSubcoreMesh(core_axis_name="c", subcore_axis_name="s")

    @functools.partial(
        pl.kernel, mesh=mesh,
        compiler_params=pltpu.CompilerParams(use_tc_tiling_on_sc=False),
        out_type=jax.ShapeDtypeStruct((total_rows, d), jnp.float32),
        scratch_types=[
            pltpu.VMEM((rows_per_chunk_idx, 128), jnp.int32),
            pltpu.VMEM((chrows, d), jnp.float32),
            pltpu.SemaphoreType.DMA,
        ],
    )
    def k3(table_hbm, idx_hbm, out_hbm, idx_v, rows_v, sem):
        wid = lax.axis_index("s") * info.num_cores + lax.axis_index("c")
        base_row = wid * b_per_w
        base_irow = wid * (b_per_w // 128)

        def chunk(ch, carry):
            pltpu.sync_copy(
                idx_hbm.at[pl.ds(base_irow + ch * rows_per_chunk_idx,
                                 rows_per_chunk_idx)], idx_v)
            handles = []
            for r in range(rows_per_chunk_idx):
                handles.append(pltpu.async_copy(
                    table_hbm.at[idx_v.at[r]],
                    rows_v.at[pl.ds(r * 128, 128)], sem))
            for h in handles:
                h.wait()
            pltpu.sync_copy(rows_v,
                            out_hbm.at[pl.ds(base_row + ch * chrows, chrows)])
            return carry

        lax.fori_loop(0, nchunks, chunk, 0)

    return k3(table, idx2d)


def kernel(states_prev, log_weights_prev, observations, controls, A, B, C):
    n, m, d = states_prev.shape
    do = observations.shape[1]
    base_key = jax.random.key(42)
    noise = jax.random.normal(jax.random.fold_in(base_key, 0),
                              states_prev.shape, jnp.float32) * 0.1

    prevT = jnp.swapaxes(states_prev, 1, 2)
    noiseT = jnp.swapaxes(noise, 1, 2)

    mblk = 4096
    nmb = m // mblk
    dc = controls.shape[1]

    spn, a3 = pl.pallas_call(
        _k1a_body,
        grid=(n, nmb),
        in_specs=[
            pl.BlockSpec((1, mblk, d), lambda j, mb: (j, mb, 0)),
            pl.BlockSpec((1, d, mblk), lambda j, mb: (j, 0, mb)),
            pl.BlockSpec((1, mblk, d), lambda j, mb: (j, mb, 0)),
            pl.BlockSpec((1, d, mblk), lambda j, mb: (j, 0, mb)),
            pl.BlockSpec((1, 1, dc), lambda j, mb: (j, 0, 0)),
            pl.BlockSpec((1, dc, 1), lambda j, mb: (j, 0, 0)),
            pl.BlockSpec(A.shape, lambda j, mb: (0, 0)),
            pl.BlockSpec(A.shape, lambda j, mb: (0, 0)),
            pl.BlockSpec(B.shape, lambda j, mb: (0, 0)),
            pl.BlockSpec((B.shape[1], B.shape[0]), lambda j, mb: (0, 0)),
            pl.BlockSpec((do, d), lambda j, mb: (0, 0)),
            pl.BlockSpec((1, do, 1), lambda j, mb: (j, 0, 0)),
            pl.BlockSpec((1, 1, mblk), lambda j, mb: (j, 0, mb)),
        ],
        out_specs=[
            pl.BlockSpec((1, mblk, d), lambda j, mb: (j, mb, 0)),
            pl.BlockSpec((1, 1, mblk), lambda j, mb: (j, 0, mb)),
        ],
        out_shape=[
            jax.ShapeDtypeStruct((n, m, d), jnp.float32),
            jax.ShapeDtypeStruct((n, 1, m), jnp.float32),
        ],
    )(states_prev, prevT, noise, noiseT,
      controls.reshape(n, 1, -1), controls.reshape(n, -1, 1), A, A.T,
      B, B.T, C.T, observations.reshape(n, do, 1),
      log_weights_prev.reshape(n, 1, m))

    lw3 = pl.pallas_call(
        _k1b_body,
        grid=(n,),
        in_specs=[pl.BlockSpec((1, 1, m), lambda j: (j, 0, 0))],
        out_specs=pl.BlockSpec((1, 1, m), lambda j: (j, 0, 0)),
        out_shape=jax.ShapeDtypeStruct((n, 1, m), jnp.float32),
    )(a3)

    best3 = pl.pallas_call(
        _k1c_body,
        grid=(n, nmb),
        in_specs=[
            pl.BlockSpec((1, 1, mblk), lambda j, mb: (j, 0, mb)),
            pl.BlockSpec((1, mblk, d), lambda j, mb: (j, mb, 0)),
        ],
        out_specs=pl.BlockSpec((1, 1, d), lambda j, mb: (j, 0, 0)),
        out_shape=jax.ShapeDtypeStruct((n, 1, d), jnp.float32),
    )(lw3, spn)

    lw = lw3.reshape(n, m)
    best_states = best3.reshape(n, d)

    lwmax = jnp.max(lw, axis=1, keepdims=True)
    mask = lw >= (lwmax - _CUTOFF)
    fits = jnp.max(jnp.sum(mask, axis=1)) <= _KMAX

    def fast_path(_):
        masked = jnp.where(mask, lw, -jnp.inf)
        vals, inds = lax.top_k(masked, _KMAX)
        w = jnp.exp(-vals)
        thr = jnp.concatenate(
            [w[:, _CH::_CH] * _E_LB,
             jnp.full((n, 1), jnp.inf, jnp.float32)], axis=1)
        karr = inds.astype(jnp.int32).reshape(n, _KMAX, 1)
        warr = w.reshape(n, _KMAX, 1)
        kb, _ = _run_k2(karr, warr, thr, n, m, _KMAX)
        return kb

    def slow_path(_):
        thr = jnp.full((n, _KMAX // _CH), -jnp.inf, jnp.float32)
        kb = jnp.full((n, m), _KBIG, jnp.int32)
        mb = jnp.full((n, m), jnp.inf, jnp.float32)
        for s in range(m // _KMAX):
            karr = (lax.broadcasted_iota(jnp.int32, (n, _KMAX, 1), 1)
                    + s * _KMAX)
            warr = jnp.exp(-lax.dynamic_slice_in_dim(lw, s * _KMAX, _KMAX,
                                                     axis=1))
            ks, ms = _run_k2(karr, warr.reshape(n, _KMAX, 1), thr, n, m,
                             _KMAX)
            upd = (ms < mb) | ((ms == mb) & (ks < kb))
            kb = jnp.where(upd, ks, kb)
            mb = jnp.where(upd, ms, mb)
        return kb

    kidx = lax.cond(fits, fast_path, slow_path, operand=None)
    gidx = kidx + jnp.arange(n, dtype=jnp.int32)[:, None] * m

    table = spn.reshape(n * m, d)
    idx2d = gidx.reshape((n * m) // 128, 128)
    states = _sc_gather(table, idx2d, n * m, d).reshape(n, m, d)

    log_weights = jnp.zeros((n, m), jnp.float32) - jnp.log(float(m))
    return best_states, states, log_weights
